# baseline (speedup 1.0000x reference)
import functools

import jax
import jax.numpy as jnp
from jax import lax
from jax.experimental import pallas as pl
from jax.experimental.pallas import tpu as pltpu
from jax.experimental.pallas import tpu_sc as plsc

_N = 10000
_E = 160000
_FEAT = 256
_HID = 512
_HEADS = 8
_HD = 64
_G = 16
_BM = 1000
_GRID = _N // _BM
_SB = 40
_NROW = _E // _SB
_P1R = _NROW // 32
_P2R = _NROW // 16
_STR = _N // 16
_F32 = jnp.float32


def _dot(a, b):
    return jnp.dot(a, b, preferred_element_type=_F32)


def _e0_body(x_ref, we_ref, be_ref, w1_ref, ac_ref,
             h_ref, xa_ref, xb_ref, xc_ref, xd_ref, a_ref):
    h = _dot(x_ref[...], we_ref[...]) + be_ref[0, :]
    h_ref[...] = h
    xw = _dot(h, w1_ref[...])
    xa_ref[...] = xw[:, 0:128]
    xb_ref[...] = xw[:, 128:256]
    xc_ref[...] = xw[:, 256:384]
    xd_ref[...] = xw[:, 384:512]
    a_ref[...] = _dot(xw, ac_ref[...])


def _e0(x, We, be, W1, acat):
    return pl.pallas_call(
        _e0_body,
        grid=(_GRID,),
        in_specs=[
            pl.BlockSpec((_BM, _FEAT), lambda i: (i, 0)),
            pl.BlockSpec((_FEAT, _HID), lambda i: (0, 0)),
            pl.BlockSpec((8, _HID), lambda i: (0, 0)),
            pl.BlockSpec((_HID, _HID), lambda i: (0, 0)),
            pl.BlockSpec((_HID, 128), lambda i: (0, 0)),
        ],
        out_specs=[
            pl.BlockSpec((_BM, _HID), lambda i: (i, 0)),
            pl.BlockSpec((_BM, 128), lambda i: (i, 0)),
            pl.BlockSpec((_BM, 128), lambda i: (i, 0)),
            pl.BlockSpec((_BM, 128), lambda i: (i, 0)),
            pl.BlockSpec((_BM, 128), lambda i: (i, 0)),
            pl.BlockSpec((_BM, 128), lambda i: (i, 0)),
        ],
        out_shape=[
            jax.ShapeDtypeStruct((_N, _HID), _F32),
            jax.ShapeDtypeStruct((_N, 128), _F32),
            jax.ShapeDtypeStruct((_N, 128), _F32),
            jax.ShapeDtypeStruct((_N, 128), _F32),
            jax.ShapeDtypeStruct((_N, 128), _F32),
            jax.ShapeDtypeStruct((_N, 128), _F32),
        ],
    )(x, We, jnp.broadcast_to(be.reshape(1, _HID), (8, _HID)), W1, acat)


def _d_body(acca_ref, accb_ref, den_ref, b_ref, sel_ref, op_ref, s_ref):
    c = pl.program_id(0)
    i = pl.program_id(1)
    acc = jnp.where(c % 2 == 1, accb_ref[...], acca_ref[...])
    den = den_ref[0] + den_ref[1]
    dexp = _dot(den, sel_ref[0]) + 1e-16
    op = acc / dexp + b_ref[0, :]
    op_ref[...] = op

    @pl.when(i == 0)
    def _():
        s_ref[...] = jnp.zeros_like(s_ref)

    s_ref[0:1, :] += jnp.sum(op, axis=0, keepdims=True)
    s_ref[1:2, :] += jnp.sum(op * op, axis=0, keepdims=True)


def _d(acca, accb, den2, b, sel):
    return pl.pallas_call(
        _d_body,
        grid=(4, _GRID),
        in_specs=[
            pl.BlockSpec((_BM, 128), lambda c, i: ((c // 2) * _GRID + i, 0)),
            pl.BlockSpec((_BM, 128), lambda c, i: ((c // 2) * _GRID + i, 0)),
            pl.BlockSpec((2, _BM, 16), lambda c, i: (0, i, 0)),
            pl.BlockSpec((8, 128), lambda c, i: (0, c)),
            pl.BlockSpec((1, 16, 128), lambda c, i: (c, 0, 0)),
        ],
        out_specs=[
            pl.BlockSpec((_BM, 128), lambda c, i: (i, c)),
            pl.BlockSpec((8, 128), lambda c, i: (0, c)),
        ],
        out_shape=[
            jax.ShapeDtypeStruct((_N, _HID), _F32),
            jax.ShapeDtypeStruct((8, _HID), _F32),
        ],
    )(acca, accb, den2, jnp.broadcast_to(b.reshape(1, _HID), (8, _HID)), sel)


def _bn_apply(op_ref, s_ref, g_ref, bt_ref, hr_ref):
    mu = s_ref[0, :] * (1.0 / _N)
    var = s_ref[1, :] * (1.0 / _N) - mu * mu
    scale = lax.rsqrt(var + 1e-5) * g_ref[0, :]
    xn = (op_ref[...] - mu) * scale + bt_ref[0, :]
    xe = jnp.where(xn > 0, xn, jnp.exp(xn) - 1.0)
    return xe + hr_ref[...]


def _e_body(op_ref, s_ref, g_ref, bt_ref, hr_ref, w_ref, ac_ref,
            h_ref, xa_ref, xb_ref, xc_ref, xd_ref, a_ref):
    h = _bn_apply(op_ref, s_ref, g_ref, bt_ref, hr_ref)
    h_ref[...] = h
    xw = _dot(h, w_ref[...])
    xa_ref[...] = xw[:, 0:128]
    xb_ref[...] = xw[:, 128:256]
    xc_ref[...] = xw[:, 256:384]
    xd_ref[...] = xw[:, 384:512]
    a_ref[...] = _dot(xw, ac_ref[...])


def _e(op, sums, gamma, beta, hres, W, acat):
    bro = lambda v: jnp.broadcast_to(v.reshape(1, _HID), (8, _HID))
    return pl.pallas_call(
        _e_body,
        grid=(_GRID,),
        in_specs=[
            pl.BlockSpec((_BM, _HID), lambda i: (i, 0)),
            pl.BlockSpec((8, _HID), lambda i: (0, 0)),
            pl.BlockSpec((8, _HID), lambda i: (0, 0)),
            pl.BlockSpec((8, _HID), lambda i: (0, 0)),
            pl.BlockSpec((_BM, _HID), lambda i: (i, 0)),
            pl.BlockSpec((_HID, _HID), lambda i: (0, 0)),
            pl.BlockSpec((_HID, 128), lambda i: (0, 0)),
        ],
        out_specs=[
            pl.BlockSpec((_BM, _HID), lambda i: (i, 0)),
            pl.BlockSpec((_BM, 128), lambda i: (i, 0)),
            pl.BlockSpec((_BM, 128), lambda i: (i, 0)),
            pl.BlockSpec((_BM, 128), lambda i: (i, 0)),
            pl.BlockSpec((_BM, 128), lambda i: (i, 0)),
            pl.BlockSpec((_BM, 128), lambda i: (i, 0)),
        ],
        out_shape=[
            jax.ShapeDtypeStruct((_N, _HID), _F32),
            jax.ShapeDtypeStruct((_N, 128), _F32),
            jax.ShapeDtypeStruct((_N, 128), _F32),
            jax.ShapeDtypeStruct((_N, 128), _F32),
            jax.ShapeDtypeStruct((_N, 128), _F32),
            jax.ShapeDtypeStruct((_N, 128), _F32),
        ],
    )(op, sums, bro(gamma), bro(beta), hres, W, acat)


def _e3_body(op_ref, s_ref, g_ref, bt_ref, hr_ref, wr1_ref, br1_ref,
             wr2_ref, batch_ref, h_ref, zn_ref, gs_ref):
    i = pl.program_id(0)
    h = _bn_apply(op_ref, s_ref, g_ref, bt_ref, hr_ref)
    h_ref[...] = h
    g1 = jnp.tanh(_dot(h, wr1_ref[...]) + br1_ref[0, :])
    gate = _dot(g1, wr2_ref[...])
    eg = jnp.exp(gate[:, 0:1])
    bb = jnp.broadcast_to(batch_ref[...], (_BM, _G))
    ii = lax.broadcasted_iota(jnp.int32, (_BM, _G), 1)
    mt = jnp.where(bb == ii, jnp.broadcast_to(eg, (_BM, _G)), 0.0)

    @pl.when(i == 0)
    def _():
        zn_ref[...] = jnp.zeros_like(zn_ref)
        gs_ref[...] = jnp.zeros_like(gs_ref)

    zn_ref[...] += lax.dot_general(
        mt, h, (((0,), (0,)), ((), ())), preferred_element_type=_F32)
    gs_ref[...] += lax.dot_general(
        mt, jnp.ones((_BM, 128), _F32), (((0,), (0,)), ((), ())),
        preferred_element_type=_F32)


def _e3(op, sums, gamma, beta, hres, Wr1, br1, Wr2p, batch2):
    bro = lambda v: jnp.broadcast_to(v.reshape(1, _HID), (8, _HID))
    return pl.pallas_call(
        _e3_body,
        grid=(_GRID,),
        in_specs=[
            pl.BlockSpec((_BM, _HID), lambda i: (i, 0)),
            pl.BlockSpec((8, _HID), lambda i: (0, 0)),
            pl.BlockSpec((8, _HID), lambda i: (0, 0)),
            pl.BlockSpec((8, _HID), lambda i: (0, 0)),
            pl.BlockSpec((_BM, _HID), lambda i: (i, 0)),
            pl.BlockSpec((_HID, _HID), lambda i: (0, 0)),
            pl.BlockSpec((8, _HID), lambda i: (0, 0)),
            pl.BlockSpec((_HID, 128), lambda i: (0, 0)),
            pl.BlockSpec((_BM, 1), lambda i: (i, 0)),
        ],
        out_specs=[
            pl.BlockSpec((_BM, _HID), lambda i: (i, 0)),
            pl.BlockSpec((_G, _HID), lambda i: (0, 0)),
            pl.BlockSpec((_G, 128), lambda i: (0, 0)),
        ],
        out_shape=[
            jax.ShapeDtypeStruct((_N, _HID), _F32),
            jax.ShapeDtypeStruct((_G, _HID), _F32),
            jax.ShapeDtypeStruct((_G, 128), _F32),
        ],
    )(op, sums, bro(gamma), bro(beta), hres, Wr1, bro(br1), Wr2p, batch2)


_MESH = plsc.VectorSubcoreMesh(core_axis_name="c", subcore_axis_name="s")


def _splat(v, lane):
    idx = (lax.iota(jnp.int32, 16) * 0 + lane)[:, None]
    return lax.gather(
        v, idx,
        lax.GatherDimensionNumbers(offset_dims=(), collapsed_slice_dims=(0,),
                                   start_index_map=(0,)),
        (1,), mode=lax.GatherScatterMode.PROMISE_IN_BOUNDS)


@functools.partial(
    pl.kernel,
    out_type=[
        jax.ShapeDtypeStruct((_E, 16), _F32),
        jax.ShapeDtypeStruct((32, _STR, 16), _F32),
    ],
    mesh=_MESH,
    scratch_types=[
        pltpu.VMEM((_P1R, _SB), jnp.int32),
        pltpu.VMEM((_P1R, _SB), jnp.int32),
        pltpu.VMEM((2, _SB, 16), _F32),
        pltpu.VMEM((2, _SB, 16), _F32),
        pltpu.VMEM((2, _SB, 16), _F32),
        pltpu.VMEM_SHARED((_N, 16), _F32),
        pltpu.SemaphoreType.DMA,
        pltpu.SemaphoreType.DMA,
        pltpu.SemaphoreType.DMA,
        pltpu.SemaphoreType.DMA,
        pltpu.SemaphoreType.DMA,
        pltpu.SemaphoreType.DMA,
    ],
    compiler_params=pltpu.CompilerParams(use_tc_tiling_on_sc=False),
)
def _sc_phase1(asrc_hbm, adst_hbm, src2_hbm, dst2_hbm, z16_hbm,
               w_hbm, den2_hbm,
               src_m, dst_m, rows_s, rows_d, w_buf, den_sp,
               ss0, ss1, sd0, sd1, sc0, sc1):
    cid = lax.axis_index("c")
    sid = lax.axis_index("s")
    wid = sid * 2 + cid
    r0 = wid * _P1R
    pltpu.sync_copy(src2_hbm.at[wid], src_m)
    pltpu.sync_copy(dst2_hbm.at[wid], dst_m)
    pltpu.sync_copy(z16_hbm.at[sid],
                    den_sp.at[pl.ds(sid * _STR, _STR)])
    plsc.subcore_barrier()

    sems = ((ss0, sd0, sc0), (ss1, sd1, sc1))

    def issue(j, t):
        pltpu.async_copy(asrc_hbm.at[src_m.at[j]], rows_s.at[t], sems[t][0])
        pltpu.async_copy(adst_hbm.at[dst_m.at[j]], rows_d.at[t], sems[t][1])

    def consume(j, t):
        pltpu.make_async_copy(asrc_hbm.at[src_m.at[j]], rows_s.at[t],
                              sems[t][0]).wait()
        pltpu.make_async_copy(adst_hbm.at[dst_m.at[j]], rows_d.at[t],
                              sems[t][1]).wait()
        for i in range(_SB):
            v = rows_s[t, i, :] + rows_d[t, i, :]
            v = jnp.where(v > 0, v, 0.2 * v)
            w_buf[t, i, :] = jnp.exp(v)
        pltpu.sync_copy(w_buf.at[t], den_sp.at[dst_m.at[j]], add=True)
        pltpu.sync_copy(w_buf.at[t], w_hbm.at[pl.ds((r0 + j) * _SB, _SB)])

        @pl.when(j + 2 < _P1R)
        def _():
            issue(j + 2, t)

    issue(0, 0)
    issue(1, 1)

    def body(m, _):
        j0 = 2 * m
        consume(j0, 0)
        consume(j0 + 1, 1)
        return 0

    lax.fori_loop(0, _P1R // 2, body, 0)
    consume(_P1R - 1, 0)
    plsc.subcore_barrier()
    pltpu.sync_copy(den_sp.at[pl.ds(sid * _STR, _STR)],
                    den2_hbm.at[cid * 16 + sid])


def _make_phase2(kc):
    @functools.partial(
        pl.kernel,
        out_type=jax.ShapeDtypeStruct((32, _STR, 128), _F32),
        mesh=_MESH,
        scratch_types=[
            pltpu.VMEM((_P2R, _SB), jnp.int32),
            pltpu.VMEM((_P2R, _SB), jnp.int32),
            pltpu.VMEM((2, _SB, 128), _F32),
            pltpu.VMEM((2, _SB, 128), _F32),
            pltpu.VMEM((2, _SB, 16), _F32),
            pltpu.VMEM_SHARED((_N, 128), _F32),
            pltpu.SemaphoreType.DMA,
            pltpu.SemaphoreType.DMA,
            pltpu.SemaphoreType.DMA,
            pltpu.SemaphoreType.DMA,
            pltpu.SemaphoreType.DMA,
            pltpu.SemaphoreType.DMA,
        ],
        compiler_params=pltpu.CompilerParams(use_tc_tiling_on_sc=False),
    )
    def _ker(tabA, tabB, w_hbm, src2_hbm, dst2_hbm, z128_hbm,
             acc_hbm,
             src_m, dst_m, rows, sbuf, w_buf, acc_sp,
             sg0, sg1, sw0, sw1, sc0, sc1):
        cid = lax.axis_index("c")
        sid = lax.axis_index("s")
        r0 = sid * _P2R
        pltpu.sync_copy(src2_hbm.at[sid], src_m)
        pltpu.sync_copy(dst2_hbm.at[sid], dst_m)
        sems = ((sg0, sw0, sc0), (sg1, sw1, sc1))
        for c_val in (0, 1):
            @pl.when(cid == c_val)
            def _():
                q = 2 * c_val + kc
                tab = tabA if c_val == 0 else tabB
                pltpu.sync_copy(z128_hbm.at[sid],
                                acc_sp.at[pl.ds(sid * _STR, _STR)])
                plsc.subcore_barrier()

                def issue(j, t):
                    pltpu.async_copy(tab.at[src_m.at[j]], rows.at[t],
                                     sems[t][0])
                    pltpu.async_copy(w_hbm.at[pl.ds((r0 + j) * _SB, _SB)],
                                     w_buf.at[t], sems[t][1])

                def consume(j, t):
                    pltpu.make_async_copy(tab.at[src_m.at[j]], rows.at[t],
                                          sems[t][0]).wait()
                    pltpu.make_async_copy(
                        w_hbm.at[pl.ds((r0 + j) * _SB, _SB)],
                        w_buf.at[t], sems[t][1]).wait()

                    @pl.when(j >= 2)
                    def _():
                        pltpu.make_async_copy(
                            sbuf.at[t], acc_sp.at[dst_m.at[j]],
                            sems[t][2]).wait()

                    for i in range(_SB):
                        wv = w_buf[t, i, :]
                        w0 = _splat(wv, 2 * q)
                        w1 = _splat(wv, 2 * q + 1)
                        for k in range(8):
                            ws = w0 if k < 4 else w1
                            sbuf[t, i, 16 * k:16 * (k + 1)] = (
                                rows[t, i, 16 * k:16 * (k + 1)] * ws)

                    @pl.when(j + 2 < _P2R)
                    def _():
                        issue(j + 2, t)

                    pltpu.async_copy(sbuf.at[t], acc_sp.at[dst_m.at[j]],
                                     sems[t][2], add=True)

                issue(0, 0)
                issue(1, 1)

                def body(m, _):
                    j0 = 2 * m
                    consume(j0, 0)
                    consume(j0 + 1, 1)
                    return 0

                lax.fori_loop(0, _P2R // 2, body, 0)
                pltpu.make_async_copy(
                    sbuf.at[0], acc_sp.at[dst_m.at[_P2R - 2]],
                    sems[0][2]).wait()
                pltpu.make_async_copy(
                    sbuf.at[1], acc_sp.at[dst_m.at[_P2R - 1]],
                    sems[1][2]).wait()
                plsc.subcore_barrier()
                pltpu.sync_copy(
                    acc_sp.at[pl.ds(sid * _STR, _STR)],
                    acc_hbm.at[c_val * 16 + sid])
                plsc.subcore_barrier()

    return _ker


_sc_phase2_a = _make_phase2(0)
_sc_phase2_b = _make_phase2(1)


def kernel(x, edge_index, batch, params):
    src_p1 = edge_index[0].reshape(32, _P1R, _SB)
    dst_p1 = edge_index[1].reshape(32, _P1R, _SB)
    src_p2 = edge_index[0].reshape(16, _P2R, _SB)
    dst_p2 = edge_index[1].reshape(16, _P2R, _SB)
    batch2 = batch.reshape(_N, 1)
    z16 = jnp.zeros((16, _STR, 16), _F32)
    z128 = jnp.zeros((16, _STR, 128), _F32)

    onehot = (jnp.arange(_HID)[:, None] // _HD
              == jnp.arange(_HEADS)[None, :]).astype(_F32)
    def _acat(lp):
        ms = lp['a_src'].reshape(_HID)[:, None] * onehot
        md = lp['a_dst'].reshape(_HID)[:, None] * onehot
        return jnp.concatenate(
            [ms, ms, md, md, jnp.zeros((_HID, 96), _F32)], axis=1)

    ci = jnp.arange(4)[:, None, None]
    hi = jnp.arange(16)[None, :, None]
    li = jnp.arange(128)[None, None, :]
    sel = (((hi == 2 * ci) & (li < 64))
           | ((hi == 2 * ci + 1) & (li >= 64))).astype(_F32)

    layers = params['layers']
    h, xa, xb, xc, xd, a128 = _e0(x, params['W_embed'], params['b_embed'],
                                  layers[0]['W'], _acat(layers[0]))
    for l in range(3):
        lp = layers[l]
        asrc = a128[:, 0:16]
        adst = a128[:, 16:32]
        w, den2 = _sc_phase1(asrc, adst, src_p1, dst_p1, z16)
        acc_a = _sc_phase2_a(xa, xc, w, src_p2, dst_p2, z128)
        acc_b = _sc_phase2_b(xb, xd, w, src_p2, dst_p2, z128)
        op, sums = _d(acc_a.reshape(2 * _N, 128), acc_b.reshape(2 * _N, 128),
                      den2.reshape(2, _N, 16), lp['b'], sel)
        if l < 2:
            nlp = layers[l + 1]
            h, xa, xb, xc, xd, a128 = _e(op, sums, lp['gamma'], lp['beta'],
                                         h, nlp['W'], _acat(nlp))
        else:
            Wr2p = jnp.pad(params['Wr2'], ((0, 0), (0, 127)))
            h, zn, gs = _e3(op, sums, lp['gamma'], lp['beta'], h,
                            params['Wr1'], params['br1'], Wr2p, batch2)
    z = zn / (gs[:, 0:1] + 1e-16)
    return h, z


# speedup vs baseline: 1.0055x; 1.0055x over previous
import functools

import jax
import jax.numpy as jnp
from jax import lax
from jax.experimental import pallas as pl
from jax.experimental.pallas import tpu as pltpu
from jax.experimental.pallas import tpu_sc as plsc

_N = 10000
_E = 160000
_FEAT = 256
_HID = 512
_HEADS = 8
_HD = 64
_G = 16
_BM = 1000
_GRID = _N // _BM
_SB = 40
_NROW = _E // _SB
_P1R = _NROW // 32
_P2R = _NROW // 16
_STR = _N // 16
_F32 = jnp.float32


def _dot(a, b):
    return jnp.dot(a, b, preferred_element_type=_F32)


def _e0_body(x_ref, we_ref, be_ref, w1_ref, ac_ref,
             h_ref, xa_ref, xb_ref, xc_ref, xd_ref, a_ref):
    h = _dot(x_ref[...], we_ref[...]) + be_ref[0, :]
    h_ref[...] = h
    xw = _dot(h, w1_ref[...])
    xa_ref[...] = xw[:, 0:128]
    xb_ref[...] = xw[:, 128:256]
    xc_ref[...] = xw[:, 256:384]
    xd_ref[...] = xw[:, 384:512]
    a_ref[...] = _dot(xw, ac_ref[...])


def _e0(x, We, be, W1, acat):
    return pl.pallas_call(
        _e0_body,
        grid=(_GRID,),
        in_specs=[
            pl.BlockSpec((_BM, _FEAT), lambda i: (i, 0)),
            pl.BlockSpec((_FEAT, _HID), lambda i: (0, 0)),
            pl.BlockSpec((8, _HID), lambda i: (0, 0)),
            pl.BlockSpec((_HID, _HID), lambda i: (0, 0)),
            pl.BlockSpec((_HID, 128), lambda i: (0, 0)),
        ],
        out_specs=[
            pl.BlockSpec((_BM, _HID), lambda i: (i, 0)),
            pl.BlockSpec((_BM, 128), lambda i: (i, 0)),
            pl.BlockSpec((_BM, 128), lambda i: (i, 0)),
            pl.BlockSpec((_BM, 128), lambda i: (i, 0)),
            pl.BlockSpec((_BM, 128), lambda i: (i, 0)),
            pl.BlockSpec((_BM, 128), lambda i: (i, 0)),
        ],
        out_shape=[
            jax.ShapeDtypeStruct((_N, _HID), _F32),
            jax.ShapeDtypeStruct((_N, 128), _F32),
            jax.ShapeDtypeStruct((_N, 128), _F32),
            jax.ShapeDtypeStruct((_N, 128), _F32),
            jax.ShapeDtypeStruct((_N, 128), _F32),
            jax.ShapeDtypeStruct((_N, 128), _F32),
        ],
    )(x, We, jnp.broadcast_to(be.reshape(1, _HID), (8, _HID)), W1, acat)


def _d_body(acca_ref, accb_ref, den_ref, b_ref, sel_ref, op_ref, s_ref):
    c = pl.program_id(0)
    i = pl.program_id(1)
    acc = jnp.where(c % 2 == 1, accb_ref[...], acca_ref[...])
    den = den_ref[0] + den_ref[1]
    dexp = _dot(den, sel_ref[0]) + 1e-16
    op = acc / dexp + b_ref[0, :]
    op_ref[...] = op

    @pl.when(i == 0)
    def _():
        s_ref[...] = jnp.zeros_like(s_ref)

    s_ref[0:1, :] += jnp.sum(op, axis=0, keepdims=True)
    s_ref[1:2, :] += jnp.sum(op * op, axis=0, keepdims=True)


def _d(acca, accb, den2, b, sel):
    return pl.pallas_call(
        _d_body,
        grid=(4, _GRID),
        in_specs=[
            pl.BlockSpec((_BM, 128), lambda c, i: ((c // 2) * _GRID + i, 0)),
            pl.BlockSpec((_BM, 128), lambda c, i: ((c // 2) * _GRID + i, 0)),
            pl.BlockSpec((2, _BM, 16), lambda c, i: (0, i, 0)),
            pl.BlockSpec((8, 128), lambda c, i: (0, c)),
            pl.BlockSpec((1, 16, 128), lambda c, i: (c, 0, 0)),
        ],
        out_specs=[
            pl.BlockSpec((_BM, 128), lambda c, i: (i, c)),
            pl.BlockSpec((8, 128), lambda c, i: (0, c)),
        ],
        out_shape=[
            jax.ShapeDtypeStruct((_N, _HID), _F32),
            jax.ShapeDtypeStruct((8, _HID), _F32),
        ],
    )(acca, accb, den2, jnp.broadcast_to(b.reshape(1, _HID), (8, _HID)), sel)


def _bn_apply(op_ref, s_ref, g_ref, bt_ref, hr_ref):
    mu = s_ref[0, :] * (1.0 / _N)
    var = s_ref[1, :] * (1.0 / _N) - mu * mu
    scale = lax.rsqrt(var + 1e-5) * g_ref[0, :]
    xn = (op_ref[...] - mu) * scale + bt_ref[0, :]
    xe = jnp.where(xn > 0, xn, jnp.exp(xn) - 1.0)
    return xe + hr_ref[...]


def _e_body(op_ref, s_ref, g_ref, bt_ref, hr_ref, w_ref, ac_ref,
            h_ref, xa_ref, xb_ref, xc_ref, xd_ref, a_ref):
    h = _bn_apply(op_ref, s_ref, g_ref, bt_ref, hr_ref)
    h_ref[...] = h
    xw = _dot(h, w_ref[...])
    xa_ref[...] = xw[:, 0:128]
    xb_ref[...] = xw[:, 128:256]
    xc_ref[...] = xw[:, 256:384]
    xd_ref[...] = xw[:, 384:512]
    a_ref[...] = _dot(xw, ac_ref[...])


def _e(op, sums, gamma, beta, hres, W, acat):
    bro = lambda v: jnp.broadcast_to(v.reshape(1, _HID), (8, _HID))
    return pl.pallas_call(
        _e_body,
        grid=(_GRID,),
        in_specs=[
            pl.BlockSpec((_BM, _HID), lambda i: (i, 0)),
            pl.BlockSpec((8, _HID), lambda i: (0, 0)),
            pl.BlockSpec((8, _HID), lambda i: (0, 0)),
            pl.BlockSpec((8, _HID), lambda i: (0, 0)),
            pl.BlockSpec((_BM, _HID), lambda i: (i, 0)),
            pl.BlockSpec((_HID, _HID), lambda i: (0, 0)),
            pl.BlockSpec((_HID, 128), lambda i: (0, 0)),
        ],
        out_specs=[
            pl.BlockSpec((_BM, _HID), lambda i: (i, 0)),
            pl.BlockSpec((_BM, 128), lambda i: (i, 0)),
            pl.BlockSpec((_BM, 128), lambda i: (i, 0)),
            pl.BlockSpec((_BM, 128), lambda i: (i, 0)),
            pl.BlockSpec((_BM, 128), lambda i: (i, 0)),
            pl.BlockSpec((_BM, 128), lambda i: (i, 0)),
        ],
        out_shape=[
            jax.ShapeDtypeStruct((_N, _HID), _F32),
            jax.ShapeDtypeStruct((_N, 128), _F32),
            jax.ShapeDtypeStruct((_N, 128), _F32),
            jax.ShapeDtypeStruct((_N, 128), _F32),
            jax.ShapeDtypeStruct((_N, 128), _F32),
            jax.ShapeDtypeStruct((_N, 128), _F32),
        ],
    )(op, sums, bro(gamma), bro(beta), hres, W, acat)


def _e3_body(op_ref, s_ref, g_ref, bt_ref, hr_ref, wr1_ref, br1_ref,
             wr2_ref, batch_ref, h_ref, zn_ref, gs_ref):
    i = pl.program_id(0)
    h = _bn_apply(op_ref, s_ref, g_ref, bt_ref, hr_ref)
    h_ref[...] = h
    g1 = jnp.tanh(_dot(h, wr1_ref[...]) + br1_ref[0, :])
    gate = _dot(g1, wr2_ref[...])
    eg = jnp.exp(gate[:, 0:1])
    bb = jnp.broadcast_to(batch_ref[...], (_BM, _G))
    ii = lax.broadcasted_iota(jnp.int32, (_BM, _G), 1)
    mt = jnp.where(bb == ii, jnp.broadcast_to(eg, (_BM, _G)), 0.0)

    @pl.when(i == 0)
    def _():
        zn_ref[...] = jnp.zeros_like(zn_ref)
        gs_ref[...] = jnp.zeros_like(gs_ref)

    zn_ref[...] += lax.dot_general(
        mt, h, (((0,), (0,)), ((), ())), preferred_element_type=_F32)
    gs_ref[...] += lax.dot_general(
        mt, jnp.ones((_BM, 128), _F32), (((0,), (0,)), ((), ())),
        preferred_element_type=_F32)


def _e3(op, sums, gamma, beta, hres, Wr1, br1, Wr2p, batch2):
    bro = lambda v: jnp.broadcast_to(v.reshape(1, _HID), (8, _HID))
    return pl.pallas_call(
        _e3_body,
        grid=(_GRID,),
        in_specs=[
            pl.BlockSpec((_BM, _HID), lambda i: (i, 0)),
            pl.BlockSpec((8, _HID), lambda i: (0, 0)),
            pl.BlockSpec((8, _HID), lambda i: (0, 0)),
            pl.BlockSpec((8, _HID), lambda i: (0, 0)),
            pl.BlockSpec((_BM, _HID), lambda i: (i, 0)),
            pl.BlockSpec((_HID, _HID), lambda i: (0, 0)),
            pl.BlockSpec((8, _HID), lambda i: (0, 0)),
            pl.BlockSpec((_HID, 128), lambda i: (0, 0)),
            pl.BlockSpec((_BM, 1), lambda i: (i, 0)),
        ],
        out_specs=[
            pl.BlockSpec((_BM, _HID), lambda i: (i, 0)),
            pl.BlockSpec((_G, _HID), lambda i: (0, 0)),
            pl.BlockSpec((_G, 128), lambda i: (0, 0)),
        ],
        out_shape=[
            jax.ShapeDtypeStruct((_N, _HID), _F32),
            jax.ShapeDtypeStruct((_G, _HID), _F32),
            jax.ShapeDtypeStruct((_G, 128), _F32),
        ],
    )(op, sums, bro(gamma), bro(beta), hres, Wr1, bro(br1), Wr2p, batch2)


_MESH = plsc.VectorSubcoreMesh(core_axis_name="c", subcore_axis_name="s")


def _splat(v, lane):
    idx = (lax.iota(jnp.int32, 16) * 0 + lane)[:, None]
    return lax.gather(
        v, idx,
        lax.GatherDimensionNumbers(offset_dims=(), collapsed_slice_dims=(0,),
                                   start_index_map=(0,)),
        (1,), mode=lax.GatherScatterMode.PROMISE_IN_BOUNDS)


@functools.partial(
    pl.kernel,
    out_type=[
        jax.ShapeDtypeStruct((_E, 16), _F32),
        jax.ShapeDtypeStruct((32, _STR, 16), _F32),
    ],
    mesh=_MESH,
    scratch_types=[
        pltpu.VMEM((_P1R, _SB), jnp.int32),
        pltpu.VMEM((_P1R, _SB), jnp.int32),
        pltpu.VMEM((2, _SB, 16), _F32),
        pltpu.VMEM((2, _SB, 16), _F32),
        pltpu.VMEM((2, _SB, 16), _F32),
        pltpu.VMEM_SHARED((_N, 16), _F32),
        pltpu.SemaphoreType.DMA,
        pltpu.SemaphoreType.DMA,
        pltpu.SemaphoreType.DMA,
        pltpu.SemaphoreType.DMA,
        pltpu.SemaphoreType.DMA,
        pltpu.SemaphoreType.DMA,
    ],
    compiler_params=pltpu.CompilerParams(use_tc_tiling_on_sc=False),
)
def _sc_phase1(asrc_hbm, adst_hbm, src2_hbm, dst2_hbm, z16_hbm,
               w_hbm, den2_hbm,
               src_m, dst_m, rows_s, rows_d, w_buf, den_sp,
               ss0, ss1, sd0, sd1, sc0, sc1):
    cid = lax.axis_index("c")
    sid = lax.axis_index("s")
    wid = sid * 2 + cid
    r0 = wid * _P1R
    pltpu.sync_copy(src2_hbm.at[wid], src_m)
    pltpu.sync_copy(dst2_hbm.at[wid], dst_m)
    pltpu.sync_copy(z16_hbm.at[sid],
                    den_sp.at[pl.ds(sid * _STR, _STR)])
    plsc.subcore_barrier()

    sems = ((ss0, sd0, sc0), (ss1, sd1, sc1))

    def issue(j, t):
        pltpu.async_copy(asrc_hbm.at[src_m.at[j]], rows_s.at[t], sems[t][0])
        pltpu.async_copy(adst_hbm.at[dst_m.at[j]], rows_d.at[t], sems[t][1])

    def consume(j, t):
        pltpu.make_async_copy(asrc_hbm.at[src_m.at[j]], rows_s.at[t],
                              sems[t][0]).wait()
        pltpu.make_async_copy(adst_hbm.at[dst_m.at[j]], rows_d.at[t],
                              sems[t][1]).wait()
        for i in range(_SB):
            v = rows_s[t, i, :] + rows_d[t, i, :]
            v = jnp.where(v > 0, v, 0.2 * v)
            w_buf[t, i, :] = jnp.exp(v)
        pltpu.sync_copy(w_buf.at[t], den_sp.at[dst_m.at[j]], add=True)
        pltpu.sync_copy(w_buf.at[t], w_hbm.at[pl.ds((r0 + j) * _SB, _SB)])

        @pl.when(j + 2 < _P1R)
        def _():
            issue(j + 2, t)

    issue(0, 0)
    issue(1, 1)

    def body(m, _):
        j0 = 2 * m
        consume(j0, 0)
        consume(j0 + 1, 1)
        return 0

    lax.fori_loop(0, _P1R // 2, body, 0)
    consume(_P1R - 1, 0)
    plsc.subcore_barrier()
    pltpu.sync_copy(den_sp.at[pl.ds(sid * _STR, _STR)],
                    den2_hbm.at[cid * 16 + sid])


def _make_phase2(kc):
    @functools.partial(
        pl.kernel,
        out_type=jax.ShapeDtypeStruct((32, _STR, 128), _F32),
        mesh=_MESH,
        scratch_types=[
            pltpu.VMEM((_P2R, _SB), jnp.int32),
            pltpu.VMEM((_P2R, _SB), jnp.int32),
            pltpu.VMEM((2, _SB, 128), _F32),
            pltpu.VMEM((2, _SB, 128), _F32),
            pltpu.VMEM((2, _SB, 16), _F32),
            pltpu.VMEM_SHARED((_N, 128), _F32),
            pltpu.SemaphoreType.DMA,
            pltpu.SemaphoreType.DMA,
            pltpu.SemaphoreType.DMA,
            pltpu.SemaphoreType.DMA,
            pltpu.SemaphoreType.DMA,
            pltpu.SemaphoreType.DMA,
        ],
        compiler_params=pltpu.CompilerParams(use_tc_tiling_on_sc=False),
    )
    def _ker(tabA, tabB, w_hbm, src2_hbm, dst2_hbm, z128_hbm,
             acc_hbm,
             src_m, dst_m, rows, sbuf, w_buf, acc_sp,
             sg0, sg1, sw0, sw1, sc0, sc1):
        cid = lax.axis_index("c")
        sid = lax.axis_index("s")
        r0 = sid * _P2R
        pltpu.sync_copy(src2_hbm.at[sid], src_m)
        pltpu.sync_copy(dst2_hbm.at[sid], dst_m)
        sems = ((sg0, sw0, sc0), (sg1, sw1, sc1))
        for c_val in (0, 1):
            @pl.when(cid == c_val)
            def _():
                q = 2 * c_val + kc
                tab = tabA if c_val == 0 else tabB
                pltpu.sync_copy(z128_hbm.at[sid],
                                acc_sp.at[pl.ds(sid * _STR, _STR)])
                plsc.subcore_barrier()

                def issue(j, t):
                    pltpu.async_copy(tab.at[src_m.at[j]], rows.at[t],
                                     sems[t][0])
                    pltpu.async_copy(w_hbm.at[pl.ds((r0 + j) * _SB, _SB)],
                                     w_buf.at[t], sems[t][1])

                def consume(j, t):
                    pltpu.make_async_copy(tab.at[src_m.at[j]], rows.at[t],
                                          sems[t][0]).wait()
                    pltpu.make_async_copy(
                        w_hbm.at[pl.ds((r0 + j) * _SB, _SB)],
                        w_buf.at[t], sems[t][1]).wait()

                    for i in range(_SB):
                        wv = w_buf[t, i, :]
                        w0 = _splat(wv, 2 * q)
                        w1 = _splat(wv, 2 * q + 1)
                        for k in range(8):
                            ws = w0 if k < 4 else w1
                            sbuf[t, i, 16 * k:16 * (k + 1)] = (
                                rows[t, i, 16 * k:16 * (k + 1)] * ws)

                    @pl.when(j + 2 < _P2R)
                    def _():
                        issue(j + 2, t)

                    pltpu.sync_copy(sbuf.at[t], acc_sp.at[dst_m.at[j]],
                                    add=True)

                issue(0, 0)
                issue(1, 1)

                def body(m, _):
                    j0 = 2 * m
                    consume(j0, 0)
                    consume(j0 + 1, 1)
                    return 0

                lax.fori_loop(0, _P2R // 2, body, 0)
                plsc.subcore_barrier()
                pltpu.sync_copy(
                    acc_sp.at[pl.ds(sid * _STR, _STR)],
                    acc_hbm.at[c_val * 16 + sid])
                plsc.subcore_barrier()

    return _ker


_sc_phase2_a = _make_phase2(0)
_sc_phase2_b = _make_phase2(1)


def kernel(x, edge_index, batch, params):
    src_p1 = edge_index[0].reshape(32, _P1R, _SB)
    dst_p1 = edge_index[1].reshape(32, _P1R, _SB)
    src_p2 = edge_index[0].reshape(16, _P2R, _SB)
    dst_p2 = edge_index[1].reshape(16, _P2R, _SB)
    batch2 = batch.reshape(_N, 1)
    z16 = jnp.zeros((16, _STR, 16), _F32)
    z128 = jnp.zeros((16, _STR, 128), _F32)

    onehot = (jnp.arange(_HID)[:, None] // _HD
              == jnp.arange(_HEADS)[None, :]).astype(_F32)
    def _acat(lp):
        ms = lp['a_src'].reshape(_HID)[:, None] * onehot
        md = lp['a_dst'].reshape(_HID)[:, None] * onehot
        return jnp.concatenate(
            [ms, ms, md, md, jnp.zeros((_HID, 96), _F32)], axis=1)

    ci = jnp.arange(4)[:, None, None]
    hi = jnp.arange(16)[None, :, None]
    li = jnp.arange(128)[None, None, :]
    sel = (((hi == 2 * ci) & (li < 64))
           | ((hi == 2 * ci + 1) & (li >= 64))).astype(_F32)

    layers = params['layers']
    h, xa, xb, xc, xd, a128 = _e0(x, params['W_embed'], params['b_embed'],
                                  layers[0]['W'], _acat(layers[0]))
    for l in range(3):
        lp = layers[l]
        asrc = a128[:, 0:16]
        adst = a128[:, 16:32]
        w, den2 = _sc_phase1(asrc, adst, src_p1, dst_p1, z16)
        acc_a = _sc_phase2_a(xa, xc, w, src_p2, dst_p2, z128)
        acc_b = _sc_phase2_b(xb, xd, w, src_p2, dst_p2, z128)
        op, sums = _d(acc_a.reshape(2 * _N, 128), acc_b.reshape(2 * _N, 128),
                      den2.reshape(2, _N, 16), lp['b'], sel)
        if l < 2:
            nlp = layers[l + 1]
            h, xa, xb, xc, xd, a128 = _e(op, sums, lp['gamma'], lp['beta'],
                                         h, nlp['W'], _acat(nlp))
        else:
            Wr2p = jnp.pad(params['Wr2'], ((0, 0), (0, 127)))
            h, zn, gs = _e3(op, sums, lp['gamma'], lp['beta'], h,
                            params['Wr1'], params['br1'], Wr2p, batch2)
    z = zn / (gs[:, 0:1] + 1e-16)
    return h, z


# speedup vs baseline: 1.0523x; 1.0465x over previous
import functools

import jax
import jax.numpy as jnp
from jax import lax
from jax.experimental import pallas as pl
from jax.experimental.pallas import tpu as pltpu
from jax.experimental.pallas import tpu_sc as plsc

_N = 10000
_E = 160000
_FEAT = 256
_HID = 512
_HEADS = 8
_HD = 64
_G = 16
_BM = 1000
_GRID = _N // _BM
_SB = 40
_NROW = _E // _SB
_P1R = _NROW // 32
_P2R = _NROW // 16
_STR = _N // 16
_F32 = jnp.float32


def _dot(a, b):
    return jnp.dot(a, b, preferred_element_type=_F32)


def _e0_body(x_ref, we_ref, be_ref, w1_ref, ac_ref,
             h_ref, xa_ref, xb_ref, xc_ref, xd_ref, a_ref):
    h = _dot(x_ref[...], we_ref[...]) + be_ref[0, :]
    h_ref[...] = h
    xw = _dot(h, w1_ref[...])
    xa_ref[...] = xw[:, 0:128]
    xb_ref[...] = xw[:, 128:256]
    xc_ref[...] = xw[:, 256:384]
    xd_ref[...] = xw[:, 384:512]
    a_ref[...] = _dot(xw, ac_ref[...])


def _e0(x, We, be, W1, acat):
    return pl.pallas_call(
        _e0_body,
        grid=(_GRID,),
        in_specs=[
            pl.BlockSpec((_BM, _FEAT), lambda i: (i, 0)),
            pl.BlockSpec((_FEAT, _HID), lambda i: (0, 0)),
            pl.BlockSpec((8, _HID), lambda i: (0, 0)),
            pl.BlockSpec((_HID, _HID), lambda i: (0, 0)),
            pl.BlockSpec((_HID, 128), lambda i: (0, 0)),
        ],
        out_specs=[
            pl.BlockSpec((_BM, _HID), lambda i: (i, 0)),
            pl.BlockSpec((_BM, 128), lambda i: (i, 0)),
            pl.BlockSpec((_BM, 128), lambda i: (i, 0)),
            pl.BlockSpec((_BM, 128), lambda i: (i, 0)),
            pl.BlockSpec((_BM, 128), lambda i: (i, 0)),
            pl.BlockSpec((_BM, 128), lambda i: (i, 0)),
        ],
        out_shape=[
            jax.ShapeDtypeStruct((_N, _HID), _F32),
            jax.ShapeDtypeStruct((_N, 128), _F32),
            jax.ShapeDtypeStruct((_N, 128), _F32),
            jax.ShapeDtypeStruct((_N, 128), _F32),
            jax.ShapeDtypeStruct((_N, 128), _F32),
            jax.ShapeDtypeStruct((_N, 128), _F32),
        ],
    )(x, We, jnp.broadcast_to(be.reshape(1, _HID), (8, _HID)), W1, acat)


def _d_body(acc_ref, den_ref, b_ref, sel_ref, op_ref, s_ref):
    i = pl.program_id(1)
    den = den_ref[0] + den_ref[1]
    dexp = _dot(den, sel_ref[0]) + 1e-16
    op = acc_ref[...] / dexp + b_ref[0, :]
    op_ref[...] = op

    @pl.when(i == 0)
    def _():
        s_ref[...] = jnp.zeros_like(s_ref)

    s_ref[0:1, :] += jnp.sum(op, axis=0, keepdims=True)
    s_ref[1:2, :] += jnp.sum(op * op, axis=0, keepdims=True)


def _d_half(acc, den2, b, sel, odd):
    return pl.pallas_call(
        _d_body,
        grid=(2, _GRID),
        in_specs=[
            pl.BlockSpec((_BM, 128), lambda g, i: (g * _GRID + i, 0)),
            pl.BlockSpec((2, _BM, 16), lambda g, i: (0, i, 0)),
            pl.BlockSpec((8, 128), lambda g, i: (0, 2 * g + odd)),
            pl.BlockSpec((1, 16, 128), lambda g, i: (2 * g + odd, 0, 0)),
        ],
        out_specs=[
            pl.BlockSpec((_BM, 128), lambda g, i: (i, g)),
            pl.BlockSpec((8, 128), lambda g, i: (0, g)),
        ],
        out_shape=[
            jax.ShapeDtypeStruct((_N, 256), _F32),
            jax.ShapeDtypeStruct((8, 256), _F32),
        ],
    )(acc, den2, jnp.broadcast_to(b.reshape(1, _HID), (8, _HID)), sel)


def _interleave(a, b):
    return jnp.concatenate(
        [a[:, 0:128], b[:, 0:128], a[:, 128:256], b[:, 128:256]], axis=1)


def _bn_apply(opa_ref, opb_ref, sa_ref, sb_ref, g_ref, bt_ref, hr_ref):
    op = _interleave(opa_ref[...], opb_ref[...])
    s0 = _interleave(sa_ref[0:1, :], sb_ref[0:1, :])[0, :]
    s1 = _interleave(sa_ref[1:2, :], sb_ref[1:2, :])[0, :]
    mu = s0 * (1.0 / _N)
    var = s1 * (1.0 / _N) - mu * mu
    scale = lax.rsqrt(var + 1e-5) * g_ref[0, :]
    xn = (op - mu) * scale + bt_ref[0, :]
    xe = jnp.where(xn > 0, xn, jnp.exp(xn) - 1.0)
    return xe + hr_ref[...]


def _e_body(opa_ref, opb_ref, sa_ref, sb_ref, g_ref, bt_ref, hr_ref,
            w_ref, ac_ref,
            h_ref, xa_ref, xb_ref, xc_ref, xd_ref, a_ref):
    h = _bn_apply(opa_ref, opb_ref, sa_ref, sb_ref, g_ref, bt_ref, hr_ref)
    h_ref[...] = h
    xw = _dot(h, w_ref[...])
    xa_ref[...] = xw[:, 0:128]
    xb_ref[...] = xw[:, 128:256]
    xc_ref[...] = xw[:, 256:384]
    xd_ref[...] = xw[:, 384:512]
    a_ref[...] = _dot(xw, ac_ref[...])


def _e(opa, opb, sa, sb, gamma, beta, hres, W, acat):
    bro = lambda v: jnp.broadcast_to(v.reshape(1, _HID), (8, _HID))
    return pl.pallas_call(
        _e_body,
        grid=(_GRID,),
        in_specs=[
            pl.BlockSpec((_BM, 256), lambda i: (i, 0)),
            pl.BlockSpec((_BM, 256), lambda i: (i, 0)),
            pl.BlockSpec((8, 256), lambda i: (0, 0)),
            pl.BlockSpec((8, 256), lambda i: (0, 0)),
            pl.BlockSpec((8, _HID), lambda i: (0, 0)),
            pl.BlockSpec((8, _HID), lambda i: (0, 0)),
            pl.BlockSpec((_BM, _HID), lambda i: (i, 0)),
            pl.BlockSpec((_HID, _HID), lambda i: (0, 0)),
            pl.BlockSpec((_HID, 128), lambda i: (0, 0)),
        ],
        out_specs=[
            pl.BlockSpec((_BM, _HID), lambda i: (i, 0)),
            pl.BlockSpec((_BM, 128), lambda i: (i, 0)),
            pl.BlockSpec((_BM, 128), lambda i: (i, 0)),
            pl.BlockSpec((_BM, 128), lambda i: (i, 0)),
            pl.BlockSpec((_BM, 128), lambda i: (i, 0)),
            pl.BlockSpec((_BM, 128), lambda i: (i, 0)),
        ],
        out_shape=[
            jax.ShapeDtypeStruct((_N, _HID), _F32),
            jax.ShapeDtypeStruct((_N, 128), _F32),
            jax.ShapeDtypeStruct((_N, 128), _F32),
            jax.ShapeDtypeStruct((_N, 128), _F32),
            jax.ShapeDtypeStruct((_N, 128), _F32),
            jax.ShapeDtypeStruct((_N, 128), _F32),
        ],
    )(opa, opb, sa, sb, bro(gamma), bro(beta), hres, W, acat)


def _e3_body(opa_ref, opb_ref, sa_ref, sb_ref, g_ref, bt_ref, hr_ref,
             wr1_ref, br1_ref, wr2_ref, batch_ref, h_ref, zn_ref, gs_ref):
    i = pl.program_id(0)
    h = _bn_apply(opa_ref, opb_ref, sa_ref, sb_ref, g_ref, bt_ref, hr_ref)
    h_ref[...] = h
    g1 = jnp.tanh(_dot(h, wr1_ref[...]) + br1_ref[0, :])
    gate = _dot(g1, wr2_ref[...])
    eg = jnp.exp(gate[:, 0:1])
    bb = jnp.broadcast_to(batch_ref[...], (_BM, _G))
    ii = lax.broadcasted_iota(jnp.int32, (_BM, _G), 1)
    mt = jnp.where(bb == ii, jnp.broadcast_to(eg, (_BM, _G)), 0.0)

    @pl.when(i == 0)
    def _():
        zn_ref[...] = jnp.zeros_like(zn_ref)
        gs_ref[...] = jnp.zeros_like(gs_ref)

    zn_ref[...] += lax.dot_general(
        mt, h, (((0,), (0,)), ((), ())), preferred_element_type=_F32)
    gs_ref[...] += lax.dot_general(
        mt, jnp.ones((_BM, 128), _F32), (((0,), (0,)), ((), ())),
        preferred_element_type=_F32)


def _e3(opa, opb, sa, sb, gamma, beta, hres, Wr1, br1, Wr2p, batch2):
    bro = lambda v: jnp.broadcast_to(v.reshape(1, _HID), (8, _HID))
    return pl.pallas_call(
        _e3_body,
        grid=(_GRID,),
        in_specs=[
            pl.BlockSpec((_BM, 256), lambda i: (i, 0)),
            pl.BlockSpec((_BM, 256), lambda i: (i, 0)),
            pl.BlockSpec((8, 256), lambda i: (0, 0)),
            pl.BlockSpec((8, 256), lambda i: (0, 0)),
            pl.BlockSpec((8, _HID), lambda i: (0, 0)),
            pl.BlockSpec((8, _HID), lambda i: (0, 0)),
            pl.BlockSpec((_BM, _HID), lambda i: (i, 0)),
            pl.BlockSpec((_HID, _HID), lambda i: (0, 0)),
            pl.BlockSpec((8, _HID), lambda i: (0, 0)),
            pl.BlockSpec((_HID, 128), lambda i: (0, 0)),
            pl.BlockSpec((_BM, 1), lambda i: (i, 0)),
        ],
        out_specs=[
            pl.BlockSpec((_BM, _HID), lambda i: (i, 0)),
            pl.BlockSpec((_G, _HID), lambda i: (0, 0)),
            pl.BlockSpec((_G, 128), lambda i: (0, 0)),
        ],
        out_shape=[
            jax.ShapeDtypeStruct((_N, _HID), _F32),
            jax.ShapeDtypeStruct((_G, _HID), _F32),
            jax.ShapeDtypeStruct((_G, 128), _F32),
        ],
    )(opa, opb, sa, sb, bro(gamma), bro(beta), hres, Wr1, bro(br1), Wr2p,
      batch2)


_MESH = plsc.VectorSubcoreMesh(core_axis_name="c", subcore_axis_name="s")


def _splat(v, lane):
    idx = (lax.iota(jnp.int32, 16) * 0 + lane)[:, None]
    return lax.gather(
        v, idx,
        lax.GatherDimensionNumbers(offset_dims=(), collapsed_slice_dims=(0,),
                                   start_index_map=(0,)),
        (1,), mode=lax.GatherScatterMode.PROMISE_IN_BOUNDS)


@functools.partial(
    pl.kernel,
    out_type=[
        jax.ShapeDtypeStruct((_E, 16), _F32),
        jax.ShapeDtypeStruct((32, _STR, 16), _F32),
    ],
    mesh=_MESH,
    scratch_types=[
        pltpu.VMEM((_P1R, _SB), jnp.int32),
        pltpu.VMEM((_P1R, _SB), jnp.int32),
        pltpu.VMEM((2, _SB, 16), _F32),
        pltpu.VMEM((2, _SB, 16), _F32),
        pltpu.VMEM((2, _SB, 16), _F32),
        pltpu.VMEM_SHARED((_N, 16), _F32),
        pltpu.SemaphoreType.DMA,
        pltpu.SemaphoreType.DMA,
        pltpu.SemaphoreType.DMA,
        pltpu.SemaphoreType.DMA,
        pltpu.SemaphoreType.DMA,
        pltpu.SemaphoreType.DMA,
    ],
    compiler_params=pltpu.CompilerParams(use_tc_tiling_on_sc=False),
)
def _sc_phase1(asrc_hbm, adst_hbm, src2_hbm, dst2_hbm, z16_hbm,
               w_hbm, den2_hbm,
               src_m, dst_m, rows_s, rows_d, w_buf, den_sp,
               ss0, ss1, sd0, sd1, sc0, sc1):
    cid = lax.axis_index("c")
    sid = lax.axis_index("s")
    wid = sid * 2 + cid
    r0 = wid * _P1R
    pltpu.sync_copy(src2_hbm.at[wid], src_m)
    pltpu.sync_copy(dst2_hbm.at[wid], dst_m)
    pltpu.sync_copy(z16_hbm.at[sid],
                    den_sp.at[pl.ds(sid * _STR, _STR)])
    plsc.subcore_barrier()

    sems = ((ss0, sd0, sc0), (ss1, sd1, sc1))

    def issue(j, t):
        pltpu.async_copy(asrc_hbm.at[src_m.at[j]], rows_s.at[t], sems[t][0])
        pltpu.async_copy(adst_hbm.at[dst_m.at[j]], rows_d.at[t], sems[t][1])

    def consume(j, t):
        pltpu.make_async_copy(asrc_hbm.at[src_m.at[j]], rows_s.at[t],
                              sems[t][0]).wait()
        pltpu.make_async_copy(adst_hbm.at[dst_m.at[j]], rows_d.at[t],
                              sems[t][1]).wait()
        for i in range(_SB):
            v = rows_s[t, i, :] + rows_d[t, i, :]
            v = jnp.where(v > 0, v, 0.2 * v)
            w_buf[t, i, :] = jnp.exp(v)
        pltpu.sync_copy(w_buf.at[t], den_sp.at[dst_m.at[j]], add=True)
        pltpu.sync_copy(w_buf.at[t], w_hbm.at[pl.ds((r0 + j) * _SB, _SB)])

        @pl.when(j + 2 < _P1R)
        def _():
            issue(j + 2, t)

    issue(0, 0)
    issue(1, 1)

    def body(m, _):
        j0 = 2 * m
        consume(j0, 0)
        consume(j0 + 1, 1)
        return 0

    lax.fori_loop(0, _P1R // 2, body, 0)
    consume(_P1R - 1, 0)
    plsc.subcore_barrier()
    pltpu.sync_copy(den_sp.at[pl.ds(sid * _STR, _STR)],
                    den2_hbm.at[cid * 16 + sid])


def _make_phase2(kc):
    @functools.partial(
        pl.kernel,
        out_type=jax.ShapeDtypeStruct((32, _STR, 128), _F32),
        mesh=_MESH,
        scratch_types=[
            pltpu.VMEM((_P2R, _SB), jnp.int32),
            pltpu.VMEM((_P2R, _SB), jnp.int32),
            pltpu.VMEM((2, _SB, 128), _F32),
            pltpu.VMEM((2, _SB, 128), _F32),
            pltpu.VMEM((2, _SB, 16), _F32),
            pltpu.VMEM_SHARED((_N, 128), _F32),
            pltpu.SemaphoreType.DMA,
            pltpu.SemaphoreType.DMA,
            pltpu.SemaphoreType.DMA,
            pltpu.SemaphoreType.DMA,
            pltpu.SemaphoreType.DMA,
            pltpu.SemaphoreType.DMA,
        ],
        compiler_params=pltpu.CompilerParams(use_tc_tiling_on_sc=False),
    )
    def _ker(tabA, tabB, w_hbm, src2_hbm, dst2_hbm, z128_hbm,
             acc_hbm,
             src_m, dst_m, rows, sbuf, w_buf, acc_sp,
             sg0, sg1, sw0, sw1, sc0, sc1):
        cid = lax.axis_index("c")
        sid = lax.axis_index("s")
        r0 = sid * _P2R
        pltpu.sync_copy(src2_hbm.at[sid], src_m)
        pltpu.sync_copy(dst2_hbm.at[sid], dst_m)
        sems = ((sg0, sw0, sc0), (sg1, sw1, sc1))
        for c_val in (0, 1):
            @pl.when(cid == c_val)
            def _():
                q = 2 * c_val + kc
                tab = tabA if c_val == 0 else tabB
                pltpu.sync_copy(z128_hbm.at[sid],
                                acc_sp.at[pl.ds(sid * _STR, _STR)])
                plsc.subcore_barrier()

                def issue(j, t):
                    pltpu.async_copy(tab.at[src_m.at[j]], rows.at[t],
                                     sems[t][0])
                    pltpu.async_copy(w_hbm.at[pl.ds((r0 + j) * _SB, _SB)],
                                     w_buf.at[t], sems[t][1])

                def consume(j, t):
                    pltpu.make_async_copy(tab.at[src_m.at[j]], rows.at[t],
                                          sems[t][0]).wait()
                    pltpu.make_async_copy(
                        w_hbm.at[pl.ds((r0 + j) * _SB, _SB)],
                        w_buf.at[t], sems[t][1]).wait()

                    for i in range(_SB):
                        wv = w_buf[t, i, :]
                        w0 = _splat(wv, 2 * q)
                        w1 = _splat(wv, 2 * q + 1)
                        for k in range(8):
                            ws = w0 if k < 4 else w1
                            sbuf[t, i, 16 * k:16 * (k + 1)] = (
                                rows[t, i, 16 * k:16 * (k + 1)] * ws)

                    @pl.when(j + 2 < _P2R)
                    def _():
                        issue(j + 2, t)

                    pltpu.sync_copy(sbuf.at[t], acc_sp.at[dst_m.at[j]],
                                    add=True)

                issue(0, 0)
                issue(1, 1)

                def body(m, _):
                    j0 = 2 * m
                    consume(j0, 0)
                    consume(j0 + 1, 1)
                    return 0

                lax.fori_loop(0, _P2R // 2, body, 0)
                plsc.subcore_barrier()
                pltpu.sync_copy(
                    acc_sp.at[pl.ds(sid * _STR, _STR)],
                    acc_hbm.at[c_val * 16 + sid])
                plsc.subcore_barrier()

    return _ker


_sc_phase2_a = _make_phase2(0)
_sc_phase2_b = _make_phase2(1)


def kernel(x, edge_index, batch, params):
    src_p1 = edge_index[0].reshape(32, _P1R, _SB)
    dst_p1 = edge_index[1].reshape(32, _P1R, _SB)
    src_p2 = edge_index[0].reshape(16, _P2R, _SB)
    dst_p2 = edge_index[1].reshape(16, _P2R, _SB)
    batch2 = batch.reshape(_N, 1)
    z16 = jnp.zeros((16, _STR, 16), _F32)
    z128 = jnp.zeros((16, _STR, 128), _F32)

    onehot = (jnp.arange(_HID)[:, None] // _HD
              == jnp.arange(_HEADS)[None, :]).astype(_F32)
    def _acat(lp):
        ms = lp['a_src'].reshape(_HID)[:, None] * onehot
        md = lp['a_dst'].reshape(_HID)[:, None] * onehot
        return jnp.concatenate(
            [ms, ms, md, md, jnp.zeros((_HID, 96), _F32)], axis=1)

    ci = jnp.arange(4)[:, None, None]
    hi = jnp.arange(16)[None, :, None]
    li = jnp.arange(128)[None, None, :]
    sel = (((hi == 2 * ci) & (li < 64))
           | ((hi == 2 * ci + 1) & (li >= 64))).astype(_F32)

    layers = params['layers']
    h, xa, xb, xc, xd, a128 = _e0(x, params['W_embed'], params['b_embed'],
                                  layers[0]['W'], _acat(layers[0]))
    for l in range(3):
        lp = layers[l]
        asrc = a128[:, 0:16]
        adst = a128[:, 16:32]
        w, den2 = _sc_phase1(asrc, adst, src_p1, dst_p1, z16)
        den2r = den2.reshape(2, _N, 16)
        acc_a = _sc_phase2_a(xa, xc, w, src_p2, dst_p2, z128)
        op_a, sums_a = _d_half(acc_a.reshape(2 * _N, 128), den2r,
                               lp['b'], sel, 0)
        acc_b = _sc_phase2_b(xb, xd, w, src_p2, dst_p2, z128)
        op_b, sums_b = _d_half(acc_b.reshape(2 * _N, 128), den2r,
                               lp['b'], sel, 1)
        if l < 2:
            nlp = layers[l + 1]
            h, xa, xb, xc, xd, a128 = _e(op_a, op_b, sums_a, sums_b,
                                         lp['gamma'], lp['beta'],
                                         h, nlp['W'], _acat(nlp))
        else:
            Wr2p = jnp.pad(params['Wr2'], ((0, 0), (0, 127)))
            h, zn, gs = _e3(op_a, op_b, sums_a, sums_b,
                            lp['gamma'], lp['beta'], h,
                            params['Wr1'], params['br1'], Wr2p, batch2)
    z = zn / (gs[:, 0:1] + 1e-16)
    return h, z
